# baseline (speedup 1.0000x reference)
import functools

import jax
import jax.numpy as jnp
from jax import lax
from jax.experimental import pallas as pl
from jax.experimental.pallas import tpu as pltpu
from jax.experimental.pallas import tpu_sc as plsc

B = 16384
D = 16
V = 1_000_000
NC = 2
NS = 16
NW = NC * NS
BPW = B // NW
CB = 128
CHUNKS = BPW // CB


@functools.partial(
    pl.kernel,
    mesh=plsc.VectorSubcoreMesh(core_axis_name="c", subcore_axis_name="s"),
    out_type=jax.ShapeDtypeStruct((B,), jnp.float32),
    compiler_params=pltpu.CompilerParams(
        needs_layout_passes=False,
        use_tc_tiling_on_sc=False,
    ),
    scratch_types=[
        pltpu.VMEM((CHUNKS, CB), jnp.int32),
        pltpu.VMEM((CHUNKS, CB), jnp.int32),
        pltpu.VMEM((D, BPW), jnp.float32),
        pltpu.VMEM((D, BPW), jnp.float32),
        pltpu.VMEM((BPW,), jnp.float32),
        pltpu.SemaphoreType.DMA,
    ],
)
def _bpr_sc(lidx_hbm, iidx_hbm, lembT_hbm, iembT_hbm, out_hbm,
            lidx_v, iidx_v, lrows_v, irows_v, out_v, sem):
    wid = lax.axis_index("s") * NC + lax.axis_index("c")
    base = wid * BPW

    pltpu.sync_copy(lidx_hbm.at[wid], lidx_v)
    pltpu.sync_copy(iidx_hbm.at[wid], iidx_v)

    copies = []
    for f in range(D):
        for j in range(CHUNKS):
            copies.append(
                pltpu.async_copy(lembT_hbm.at[f].at[lidx_v.at[j]],
                                 lrows_v.at[f, pl.ds(j * CB, CB)], sem))
            copies.append(
                pltpu.async_copy(iembT_hbm.at[f].at[iidx_v.at[j]],
                                 irows_v.at[f, pl.ds(j * CB, CB)], sem))
    for c in copies:
        c.wait()

    def block(t, carry):
        r0 = t * 16
        acc = None
        for f in range(D):
            p = lrows_v[f, pl.ds(r0, 16)] * irows_v[f, pl.ds(r0, 16)]
            acc = p if acc is None else acc + p
        out_v[pl.ds(r0, 16)] = acc
        return carry

    lax.fori_loop(0, BPW // 16, block, 0)

    pltpu.sync_copy(out_v, out_hbm.at[pl.ds(base, BPW)])


def kernel(user_pos_indices, user_neg_indices, list_indices, item_indices,
           list_emb, item_emb):
    lidx = list_indices.astype(jnp.int32).reshape(NW, CHUNKS, CB)
    iidx = item_indices.astype(jnp.int32).reshape(NW, CHUNKS, CB)
    return _bpr_sc(lidx, iidx, list_emb.T, item_emb.T)
